# Initial kernel scaffold; baseline (speedup 1.0000x reference)
import jax
import jax.numpy as jnp
from jax.experimental import pallas as pl


def kernel(x, edge_index, W1, b1, W2, b2, W3, b3):
    raise NotImplementedError("write your pallas kernel here")


# speedup vs baseline: 5.2913x; 5.2913x over previous
import functools

import jax
import jax.numpy as jnp
from jax import lax
from jax.experimental import pallas as pl
from jax.experimental.pallas import tpu as pltpu
from jax.experimental.pallas import tpu_sc as plsc

N = 10000
NPAD = 10240
E = 320000
CHUNK = 128
NW = 32
CHUNKS_PER_TILE = 80
EPAD = NW * CHUNKS_PER_TILE * CHUNK
ROWS_PER_TILE = NPAD // 16

_mesh = plsc.VectorSubcoreMesh(core_axis_name="c", subcore_axis_name="s")


def _deg_kernel(dst_hbm, d0_hbm, d1_hbm, dacc, zb, ones_v, dst_v):
    c = lax.axis_index("c")
    s = lax.axis_index("s")
    wid = s * 2 + c

    def zfill(j, carry):
        zb[pl.ds(j * 16, 16)] = jnp.zeros((16,), jnp.float32)
        return carry
    lax.fori_loop(0, ROWS_PER_TILE // 16, zfill, 0)

    def ofill(j, carry):
        ones_v[pl.ds(j * 16, 16)] = jnp.ones((16,), jnp.float32)
        return carry
    lax.fori_loop(0, CHUNK // 16, ofill, 0)

    rbase = s * ROWS_PER_TILE
    pltpu.sync_copy(zb, dacc.at[pl.ds(rbase, ROWS_PER_TILE)])
    plsc.subcore_barrier()

    pltpu.sync_copy(dst_hbm.at[pl.ds(wid * CHUNKS_PER_TILE, CHUNKS_PER_TILE)],
                    dst_v)

    def body(j, carry):
        pltpu.sync_copy(ones_v, dacc.at[dst_v.at[j]], add=True)
        return carry
    lax.fori_loop(0, CHUNKS_PER_TILE, body, 0)

    plsc.subcore_barrier()

    @pl.when(c == 0)
    def _():
        pltpu.sync_copy(dacc.at[pl.ds(rbase, ROWS_PER_TILE)],
                        d0_hbm.at[pl.ds(rbase, ROWS_PER_TILE)])

    @pl.when(c == 1)
    def _():
        pltpu.sync_copy(dacc.at[pl.ds(rbase, ROWS_PER_TILE)],
                        d1_hbm.at[pl.ds(rbase, ROWS_PER_TILE)])


def _degrees(dst2):
    k = functools.partial(
        pl.kernel,
        mesh=_mesh,
        out_type=[jax.ShapeDtypeStruct((NPAD,), jnp.float32),
                  jax.ShapeDtypeStruct((NPAD,), jnp.float32)],
        scratch_types=[
            pltpu.VMEM_SHARED((NPAD,), jnp.float32),
            pltpu.VMEM((ROWS_PER_TILE,), jnp.float32),
            pltpu.VMEM((CHUNK,), jnp.float32),
            pltpu.VMEM((CHUNKS_PER_TILE, CHUNK), jnp.int32),
        ],
    )(_deg_kernel)
    return k(dst2)


def _prop_kernel(y_hbm, src_hbm, dst_hbm, p0_hbm, p1_hbm,
                 acc, src_v, dst_v, rows_v, gsem):
    c = lax.axis_index("c")
    s = lax.axis_index("s")
    wid = s * 2 + c
    rbase = s * ROWS_PER_TILE

    pltpu.sync_copy(y_hbm.at[pl.ds(rbase, ROWS_PER_TILE)],
                    acc.at[pl.ds(rbase, ROWS_PER_TILE)])
    pltpu.sync_copy(src_hbm.at[pl.ds(wid * CHUNKS_PER_TILE, CHUNKS_PER_TILE)],
                    src_v)
    pltpu.sync_copy(dst_hbm.at[pl.ds(wid * CHUNKS_PER_TILE, CHUNKS_PER_TILE)],
                    dst_v)
    plsc.subcore_barrier()

    def body(j, carry):
        pltpu.async_copy(y_hbm.at[src_v.at[j]], rows_v, gsem).wait()
        pltpu.sync_copy(rows_v, acc.at[dst_v.at[j]], add=True)
        return carry
    lax.fori_loop(0, CHUNKS_PER_TILE, body, 0)

    plsc.subcore_barrier()

    @pl.when(c == 0)
    def _():
        pltpu.sync_copy(acc.at[pl.ds(rbase, ROWS_PER_TILE)],
                        p0_hbm.at[pl.ds(rbase, ROWS_PER_TILE)])

    @pl.when(c == 1)
    def _():
        pltpu.sync_copy(acc.at[pl.ds(rbase, ROWS_PER_TILE)],
                        p1_hbm.at[pl.ds(rbase, ROWS_PER_TILE)])


def _propagate(y, src2, dst2):
    k = functools.partial(
        pl.kernel,
        mesh=_mesh,
        out_type=[jax.ShapeDtypeStruct((NPAD, 128), jnp.float32),
                  jax.ShapeDtypeStruct((NPAD, 128), jnp.float32)],
        scratch_types=[
            pltpu.VMEM_SHARED((NPAD, 128), jnp.float32),
            pltpu.VMEM((CHUNKS_PER_TILE, CHUNK), jnp.int32),
            pltpu.VMEM((CHUNKS_PER_TILE, CHUNK), jnp.int32),
            pltpu.VMEM((CHUNK, 128), jnp.float32),
            pltpu.SemaphoreType.DMA,
        ],
    )(_prop_kernel)
    return k(y, src2, dst2)


_ROWS = 1024
_GRID = NPAD // _ROWS


def _first_body(x_ref, w_ref, d0_ref, d1_ref, y_ref, dinv_ref):
    dinv = lax.rsqrt(d0_ref[...] + d1_ref[...] + 1.0)
    y_ref[...] = dinv * jnp.dot(x_ref[...], w_ref[...],
                                preferred_element_type=jnp.float32,
                                precision=lax.Precision.HIGHEST)
    dinv_ref[...] = dinv


def _first(x_pad, W1, d0, d1):
    rb = lambda i: (i, 0)
    return pl.pallas_call(
        _first_body,
        grid=(_GRID,),
        in_specs=[
            pl.BlockSpec((_ROWS, 128), rb),
            pl.BlockSpec((128, 128), lambda i: (0, 0)),
            pl.BlockSpec((_ROWS, 1), rb),
            pl.BlockSpec((_ROWS, 1), rb),
        ],
        out_specs=[pl.BlockSpec((_ROWS, 128), rb),
                   pl.BlockSpec((_ROWS, 1), rb)],
        out_shape=[jax.ShapeDtypeStruct((NPAD, 128), jnp.float32),
                   jax.ShapeDtypeStruct((NPAD, 1), jnp.float32)],
    )(x_pad, W1, d0.reshape(NPAD, 1), d1.reshape(NPAD, 1))


def _comb1_body(p0, p1, y, dinv, w_ref, b_ref, out_ref):
    s = p0[...] + p1[...] - y[...]
    h = jnp.maximum(dinv[...] * s + b_ref[...], 0.0)
    out_ref[...] = dinv[...] * jnp.dot(h, w_ref[...],
                                       preferred_element_type=jnp.float32,
                                       precision=lax.Precision.HIGHEST)


def _comb1(p0, p1, y1, dinv, W2, b1):
    rb = lambda i: (i, 0)
    return pl.pallas_call(
        _comb1_body,
        grid=(_GRID,),
        in_specs=[
            pl.BlockSpec((_ROWS, 128), rb),
            pl.BlockSpec((_ROWS, 128), rb),
            pl.BlockSpec((_ROWS, 128), rb),
            pl.BlockSpec((_ROWS, 1), rb),
            pl.BlockSpec((128, 256), lambda i: (0, 0)),
            pl.BlockSpec((1, 128), lambda i: (0, 0)),
        ],
        out_specs=pl.BlockSpec((_ROWS, 256), rb),
        out_shape=jax.ShapeDtypeStruct((NPAD, 256), jnp.float32),
    )(p0, p1, y1, dinv, W2, b1.reshape(1, 128))


def _comb2_body(qa0, qa1, qb0, qb1, y2, dinv, w_ref, b_ref, out_ref):
    sa = qa0[...] + qa1[...] - y2[:, :128]
    sb = qb0[...] + qb1[...] - y2[:, 128:]
    s = jnp.concatenate([sa, sb], axis=1)
    h = jnp.maximum(dinv[...] * s + b_ref[...], 0.0)
    out_ref[...] = dinv[...] * jnp.dot(h, w_ref[...],
                                       preferred_element_type=jnp.float32,
                                       precision=lax.Precision.HIGHEST)


def _comb2(qa0, qa1, qb0, qb1, y2, dinv, W3, b2):
    rb = lambda i: (i, 0)
    return pl.pallas_call(
        _comb2_body,
        grid=(_GRID,),
        in_specs=[
            pl.BlockSpec((_ROWS, 128), rb),
            pl.BlockSpec((_ROWS, 128), rb),
            pl.BlockSpec((_ROWS, 128), rb),
            pl.BlockSpec((_ROWS, 128), rb),
            pl.BlockSpec((_ROWS, 256), rb),
            pl.BlockSpec((_ROWS, 1), rb),
            pl.BlockSpec((256, 128), lambda i: (0, 0)),
            pl.BlockSpec((1, 256), lambda i: (0, 0)),
        ],
        out_specs=pl.BlockSpec((_ROWS, 128), rb),
        out_shape=jax.ShapeDtypeStruct((NPAD, 128), jnp.float32),
    )(qa0, qa1, qb0, qb1, y2, dinv, W3, b2.reshape(1, 256))


def _final_body(r0, r1, y3, dinv, b_ref, out_ref):
    out_ref[...] = dinv[...] * (r0[...] + r1[...] - y3[...]) + b_ref[...]


def _final(r0, r1, y3, dinv, b3):
    rb = lambda i: (i, 0)
    rows = 1000
    return pl.pallas_call(
        _final_body,
        grid=(N // rows,),
        in_specs=[
            pl.BlockSpec((rows, 128), rb),
            pl.BlockSpec((rows, 128), rb),
            pl.BlockSpec((rows, 128), rb),
            pl.BlockSpec((rows, 1), rb),
            pl.BlockSpec((1, 128), lambda i: (0, 0)),
        ],
        out_specs=pl.BlockSpec((rows, 128), rb),
        out_shape=jax.ShapeDtypeStruct((N, 128), jnp.float32),
    )(r0, r1, y3, dinv, b3.reshape(1, 128))


def kernel(x, edge_index, W1, b1, W2, b2, W3, b3):
    src = edge_index[0].astype(jnp.int32)
    dst = edge_index[1].astype(jnp.int32)
    pad = EPAD - E
    srcp = jnp.concatenate([src, jnp.zeros((pad,), jnp.int32)])
    dstp = jnp.concatenate([dst, jnp.full((pad,), N, jnp.int32)])
    src2 = srcp.reshape(EPAD // CHUNK, CHUNK)
    dst2 = dstp.reshape(EPAD // CHUNK, CHUNK)
    x_pad = jnp.pad(x, ((0, NPAD - N), (0, 0)))

    d0, d1 = _degrees(dst2)
    y1, dinv = _first(x_pad, W1, d0, d1)

    p0, p1 = _propagate(y1, src2, dst2)
    y2 = _comb1(p0, p1, y1, dinv, W2, b1)

    qa0, qa1 = _propagate(y2[:, :128], src2, dst2)
    qb0, qb1 = _propagate(y2[:, 128:], src2, dst2)
    y3 = _comb2(qa0, qa1, qb0, qb1, y2, dinv, W3, b2)

    r0, r1 = _propagate(y3, src2, dst2)
    return _final(r0, r1, y3, dinv, b3)


# speedup vs baseline: 20.1973x; 3.8171x over previous
import functools

import jax
import jax.numpy as jnp
from jax import lax
from jax.experimental import pallas as pl
from jax.experimental.pallas import tpu as pltpu
from jax.experimental.pallas import tpu_sc as plsc

N = 10000
NPAD = 10240
E = 320000
CHUNK = 128
NW = 32
CHUNKS_PER_TILE = 80
GK = 16
EPAD = NW * CHUNKS_PER_TILE * CHUNK
ROWS_PER_TILE = NPAD // 16

_mesh = plsc.VectorSubcoreMesh(core_axis_name="c", subcore_axis_name="s")


def _deg_kernel(dst_hbm, d0_hbm, d1_hbm, dacc, zb, ones_v, dst_v):
    c = lax.axis_index("c")
    s = lax.axis_index("s")
    wid = s * 2 + c

    def zfill(j, carry):
        zb[pl.ds(j * 16, 16)] = jnp.zeros((16,), jnp.float32)
        return carry
    lax.fori_loop(0, ROWS_PER_TILE // 16, zfill, 0)

    def ofill(j, carry):
        ones_v[pl.ds(j * 16, 16)] = jnp.ones((16,), jnp.float32)
        return carry
    lax.fori_loop(0, CHUNK // 16, ofill, 0)

    rbase = s * ROWS_PER_TILE
    pltpu.sync_copy(zb, dacc.at[pl.ds(rbase, ROWS_PER_TILE)])
    plsc.subcore_barrier()

    pltpu.sync_copy(dst_hbm.at[pl.ds(wid * CHUNKS_PER_TILE, CHUNKS_PER_TILE)],
                    dst_v)

    def body(j, carry):
        pltpu.sync_copy(ones_v, dacc.at[dst_v.at[j]], add=True)
        return carry
    lax.fori_loop(0, CHUNKS_PER_TILE, body, 0)

    plsc.subcore_barrier()

    @pl.when(c == 0)
    def _():
        pltpu.sync_copy(dacc.at[pl.ds(rbase, ROWS_PER_TILE)],
                        d0_hbm.at[pl.ds(rbase, ROWS_PER_TILE)])

    @pl.when(c == 1)
    def _():
        pltpu.sync_copy(dacc.at[pl.ds(rbase, ROWS_PER_TILE)],
                        d1_hbm.at[pl.ds(rbase, ROWS_PER_TILE)])


def _degrees(dst2):
    k = functools.partial(
        pl.kernel,
        mesh=_mesh,
        out_type=[jax.ShapeDtypeStruct((NPAD,), jnp.float32),
                  jax.ShapeDtypeStruct((NPAD,), jnp.float32)],
        scratch_types=[
            pltpu.VMEM_SHARED((NPAD,), jnp.float32),
            pltpu.VMEM((ROWS_PER_TILE,), jnp.float32),
            pltpu.VMEM((CHUNK,), jnp.float32),
            pltpu.VMEM((CHUNKS_PER_TILE, CHUNK), jnp.int32),
        ],
    )(_deg_kernel)
    return k(dst2)


def _prop_kernel(y_hbm, src_hbm, dst_hbm, p0_hbm, p1_hbm,
                 acc, src_v, dst_v, rows0, rows1, gsem0, gsem1):
    c = lax.axis_index("c")
    s = lax.axis_index("s")
    wid = s * 2 + c
    rbase = s * ROWS_PER_TILE

    pltpu.sync_copy(y_hbm.at[pl.ds(rbase, ROWS_PER_TILE)],
                    acc.at[pl.ds(rbase, ROWS_PER_TILE)])
    plsc.subcore_barrier()

    def group(g, carry):
        base = wid * CHUNKS_PER_TILE + g * GK
        pltpu.sync_copy(src_hbm.at[pl.ds(base, GK)], src_v)
        pltpu.sync_copy(dst_hbm.at[pl.ds(base, GK)], dst_v)
        pltpu.async_copy(y_hbm.at[src_v.at[0]], rows0, gsem0)

        def body(i, carry2):
            j0 = 2 * i
            pltpu.make_async_copy(y_hbm.at[src_v.at[j0]], rows0, gsem0).wait()
            pltpu.async_copy(y_hbm.at[src_v.at[j0 + 1]], rows1, gsem1)
            pltpu.sync_copy(rows0, acc.at[dst_v.at[j0]], add=True)
            pltpu.make_async_copy(y_hbm.at[src_v.at[j0 + 1]], rows1,
                                  gsem1).wait()

            @pl.when(j0 + 2 < GK)
            def _():
                pltpu.async_copy(y_hbm.at[src_v.at[j0 + 2]], rows0, gsem0)
            pltpu.sync_copy(rows1, acc.at[dst_v.at[j0 + 1]], add=True)
            return carry2
        lax.fori_loop(0, GK // 2, body, 0)
        return carry
    lax.fori_loop(0, CHUNKS_PER_TILE // GK, group, 0)

    plsc.subcore_barrier()

    @pl.when(c == 0)
    def _():
        pltpu.sync_copy(acc.at[pl.ds(rbase, ROWS_PER_TILE)],
                        p0_hbm.at[pl.ds(rbase, ROWS_PER_TILE)])

    @pl.when(c == 1)
    def _():
        pltpu.sync_copy(acc.at[pl.ds(rbase, ROWS_PER_TILE)],
                        p1_hbm.at[pl.ds(rbase, ROWS_PER_TILE)])


def _prop2_kernel(ya_hbm, yb_hbm, src_hbm, dst_hbm, pa_hbm, pb_hbm,
                  acc, src_v, dst_v, rows0, rows1, gsem0, gsem1):
    c = lax.axis_index("c")
    s = lax.axis_index("s")
    rbase = s * ROWS_PER_TILE
    cpt2 = 2 * CHUNKS_PER_TILE

    def run(y_hbm, p_hbm):
        pltpu.sync_copy(y_hbm.at[pl.ds(rbase, ROWS_PER_TILE)],
                        acc.at[pl.ds(rbase, ROWS_PER_TILE)])
        plsc.subcore_barrier()

        def group(g, carry):
            base = s * cpt2 + g * GK
            pltpu.sync_copy(src_hbm.at[pl.ds(base, GK)], src_v)
            pltpu.sync_copy(dst_hbm.at[pl.ds(base, GK)], dst_v)
            pltpu.async_copy(y_hbm.at[src_v.at[0]], rows0, gsem0)

            def body(i, carry2):
                j0 = 2 * i
                pltpu.make_async_copy(y_hbm.at[src_v.at[j0]], rows0,
                                      gsem0).wait()
                pltpu.async_copy(y_hbm.at[src_v.at[j0 + 1]], rows1, gsem1)
                pltpu.sync_copy(rows0, acc.at[dst_v.at[j0]], add=True)
                pltpu.make_async_copy(y_hbm.at[src_v.at[j0 + 1]], rows1,
                                      gsem1).wait()

                @pl.when(j0 + 2 < GK)
                def _():
                    pltpu.async_copy(y_hbm.at[src_v.at[j0 + 2]], rows0, gsem0)
                pltpu.sync_copy(rows1, acc.at[dst_v.at[j0 + 1]], add=True)
                return carry2
            lax.fori_loop(0, GK // 2, body, 0)
            return carry
        lax.fori_loop(0, cpt2 // GK, group, 0)

        plsc.subcore_barrier()
        pltpu.sync_copy(acc.at[pl.ds(rbase, ROWS_PER_TILE)],
                        p_hbm.at[pl.ds(rbase, ROWS_PER_TILE)])

    @pl.when(c == 0)
    def _():
        run(ya_hbm, pa_hbm)

    @pl.when(c == 1)
    def _():
        run(yb_hbm, pb_hbm)


def _propagate2(ya, yb, src2, dst2):
    k = functools.partial(
        pl.kernel,
        mesh=_mesh,
        out_type=[jax.ShapeDtypeStruct((NPAD, 128), jnp.float32),
                  jax.ShapeDtypeStruct((NPAD, 128), jnp.float32)],
        scratch_types=[
            pltpu.VMEM_SHARED((NPAD, 128), jnp.float32),
            pltpu.VMEM((GK, CHUNK), jnp.int32),
            pltpu.VMEM((GK, CHUNK), jnp.int32),
            pltpu.VMEM((CHUNK, 128), jnp.float32),
            pltpu.VMEM((CHUNK, 128), jnp.float32),
            pltpu.SemaphoreType.DMA,
            pltpu.SemaphoreType.DMA,
        ],
    )(_prop2_kernel)
    return k(ya, yb, src2, dst2)


def _propagate(y, src2, dst2):
    k = functools.partial(
        pl.kernel,
        mesh=_mesh,
        out_type=[jax.ShapeDtypeStruct((NPAD, 128), jnp.float32),
                  jax.ShapeDtypeStruct((NPAD, 128), jnp.float32)],
        scratch_types=[
            pltpu.VMEM_SHARED((NPAD, 128), jnp.float32),
            pltpu.VMEM((GK, CHUNK), jnp.int32),
            pltpu.VMEM((GK, CHUNK), jnp.int32),
            pltpu.VMEM((CHUNK, 128), jnp.float32),
            pltpu.VMEM((CHUNK, 128), jnp.float32),
            pltpu.SemaphoreType.DMA,
            pltpu.SemaphoreType.DMA,
        ],
    )(_prop_kernel)
    return k(y, src2, dst2)


_ROWS = 1024
_GRID = NPAD // _ROWS


def _first_body(x_ref, w_ref, d0_ref, d1_ref, y_ref, dinv_ref):
    dinv = lax.rsqrt(d0_ref[...] + d1_ref[...] + 1.0)
    y_ref[...] = dinv * jnp.dot(x_ref[...], w_ref[...],
                                preferred_element_type=jnp.float32,
                                precision=lax.Precision.HIGHEST)
    dinv_ref[...] = dinv


def _first(x_pad, W1, d0, d1):
    rb = lambda i: (i, 0)
    return pl.pallas_call(
        _first_body,
        grid=(_GRID,),
        in_specs=[
            pl.BlockSpec((_ROWS, 128), rb),
            pl.BlockSpec((128, 128), lambda i: (0, 0)),
            pl.BlockSpec((_ROWS, 1), rb),
            pl.BlockSpec((_ROWS, 1), rb),
        ],
        out_specs=[pl.BlockSpec((_ROWS, 128), rb),
                   pl.BlockSpec((_ROWS, 1), rb)],
        out_shape=[jax.ShapeDtypeStruct((NPAD, 128), jnp.float32),
                   jax.ShapeDtypeStruct((NPAD, 1), jnp.float32)],
    )(x_pad, W1, d0.reshape(NPAD, 1), d1.reshape(NPAD, 1))


def _comb1_body(p0, p1, y, dinv, w_ref, b_ref, out_ref):
    s = p0[...] + p1[...] - y[...]
    h = jnp.maximum(dinv[...] * s + b_ref[...], 0.0)
    out_ref[...] = dinv[...] * jnp.dot(h, w_ref[...],
                                       preferred_element_type=jnp.float32,
                                       precision=lax.Precision.HIGHEST)


def _comb1(p0, p1, y1, dinv, W2, b1):
    rb = lambda i: (i, 0)
    return pl.pallas_call(
        _comb1_body,
        grid=(_GRID,),
        in_specs=[
            pl.BlockSpec((_ROWS, 128), rb),
            pl.BlockSpec((_ROWS, 128), rb),
            pl.BlockSpec((_ROWS, 128), rb),
            pl.BlockSpec((_ROWS, 1), rb),
            pl.BlockSpec((128, 256), lambda i: (0, 0)),
            pl.BlockSpec((1, 128), lambda i: (0, 0)),
        ],
        out_specs=pl.BlockSpec((_ROWS, 256), rb),
        out_shape=jax.ShapeDtypeStruct((NPAD, 256), jnp.float32),
    )(p0, p1, y1, dinv, W2, b1.reshape(1, 128))


def _comb2_body(qa, qb, dinv, w_ref, b_ref, out_ref):
    s = jnp.concatenate([qa[...], qb[...]], axis=1)
    h = jnp.maximum(dinv[...] * s + b_ref[...], 0.0)
    out_ref[...] = dinv[...] * jnp.dot(h, w_ref[...],
                                       preferred_element_type=jnp.float32,
                                       precision=lax.Precision.HIGHEST)


def _comb2(qa, qb, dinv, W3, b2):
    rb = lambda i: (i, 0)
    return pl.pallas_call(
        _comb2_body,
        grid=(_GRID,),
        in_specs=[
            pl.BlockSpec((_ROWS, 128), rb),
            pl.BlockSpec((_ROWS, 128), rb),
            pl.BlockSpec((_ROWS, 1), rb),
            pl.BlockSpec((256, 128), lambda i: (0, 0)),
            pl.BlockSpec((1, 256), lambda i: (0, 0)),
        ],
        out_specs=pl.BlockSpec((_ROWS, 128), rb),
        out_shape=jax.ShapeDtypeStruct((NPAD, 128), jnp.float32),
    )(qa, qb, dinv, W3, b2.reshape(1, 256))


def _final_body(r0, r1, y3, dinv, b_ref, out_ref):
    out_ref[...] = dinv[...] * (r0[...] + r1[...] - y3[...]) + b_ref[...]


def _final(r0, r1, y3, dinv, b3):
    rb = lambda i: (i, 0)
    rows = 1000
    return pl.pallas_call(
        _final_body,
        grid=(N // rows,),
        in_specs=[
            pl.BlockSpec((rows, 128), rb),
            pl.BlockSpec((rows, 128), rb),
            pl.BlockSpec((rows, 128), rb),
            pl.BlockSpec((rows, 1), rb),
            pl.BlockSpec((1, 128), lambda i: (0, 0)),
        ],
        out_specs=pl.BlockSpec((rows, 128), rb),
        out_shape=jax.ShapeDtypeStruct((N, 128), jnp.float32),
    )(r0, r1, y3, dinv, b3.reshape(1, 128))


def kernel(x, edge_index, W1, b1, W2, b2, W3, b3):
    src = edge_index[0].astype(jnp.int32)
    dst = edge_index[1].astype(jnp.int32)
    pad = EPAD - E
    padi = jnp.arange(pad, dtype=jnp.int32)
    srcp = jnp.concatenate([src, padi % N])
    dstp = jnp.concatenate([dst, N + padi % (NPAD - N)])
    src2 = srcp.reshape(EPAD // CHUNK, CHUNK)
    dst2 = dstp.reshape(EPAD // CHUNK, CHUNK)
    x_pad = jnp.pad(x, ((0, NPAD - N), (0, 0)))

    d0, d1 = _degrees(dst2)
    y1, dinv = _first(x_pad, W1, d0, d1)

    p0, p1 = _propagate(y1, src2, dst2)
    y2 = _comb1(p0, p1, y1, dinv, W2, b1)

    qa, qb = _propagate2(y2[:, :128], y2[:, 128:], src2, dst2)
    y3 = _comb2(qa, qb, dinv, W3, b2)

    r0, r1 = _propagate(y3, src2, dst2)
    return _final(r0, r1, y3, dinv, b3)


# speedup vs baseline: 21.1549x; 1.0474x over previous
import functools

import jax
import jax.numpy as jnp
from jax import lax
from jax.experimental import pallas as pl
from jax.experimental.pallas import tpu as pltpu
from jax.experimental.pallas import tpu_sc as plsc

N = 10000
NPAD = 10240
E = 320000
CHUNK = 128
NW = 32
CHUNKS_PER_TILE = 80
GK = 40
EPAD = NW * CHUNKS_PER_TILE * CHUNK
ROWS_PER_TILE = NPAD // 16

_mesh = plsc.VectorSubcoreMesh(core_axis_name="c", subcore_axis_name="s")


def _deg_kernel(dst_hbm, d0_hbm, d1_hbm, dacc, zb, ones_v, dst_v):
    c = lax.axis_index("c")
    s = lax.axis_index("s")
    wid = s * 2 + c

    def zfill(j, carry):
        zb[pl.ds(j * 16, 16)] = jnp.zeros((16,), jnp.float32)
        return carry
    lax.fori_loop(0, ROWS_PER_TILE // 16, zfill, 0)

    def ofill(j, carry):
        ones_v[pl.ds(j * 16, 16)] = jnp.ones((16,), jnp.float32)
        return carry
    lax.fori_loop(0, CHUNK // 16, ofill, 0)

    rbase = s * ROWS_PER_TILE
    pltpu.sync_copy(zb, dacc.at[pl.ds(rbase, ROWS_PER_TILE)])
    plsc.subcore_barrier()

    pltpu.sync_copy(dst_hbm.at[pl.ds(wid * CHUNKS_PER_TILE, CHUNKS_PER_TILE)],
                    dst_v)

    def body(j, carry):
        pltpu.sync_copy(ones_v, dacc.at[dst_v.at[j]], add=True)
        return carry
    lax.fori_loop(0, CHUNKS_PER_TILE, body, 0)

    plsc.subcore_barrier()

    @pl.when(c == 0)
    def _():
        pltpu.sync_copy(dacc.at[pl.ds(rbase, ROWS_PER_TILE)],
                        d0_hbm.at[pl.ds(rbase, ROWS_PER_TILE)])

    @pl.when(c == 1)
    def _():
        pltpu.sync_copy(dacc.at[pl.ds(rbase, ROWS_PER_TILE)],
                        d1_hbm.at[pl.ds(rbase, ROWS_PER_TILE)])


def _degrees(dst2):
    k = functools.partial(
        pl.kernel,
        mesh=_mesh,
        out_type=[jax.ShapeDtypeStruct((NPAD,), jnp.float32),
                  jax.ShapeDtypeStruct((NPAD,), jnp.float32)],
        scratch_types=[
            pltpu.VMEM_SHARED((NPAD,), jnp.float32),
            pltpu.VMEM((ROWS_PER_TILE,), jnp.float32),
            pltpu.VMEM((CHUNK,), jnp.float32),
            pltpu.VMEM((CHUNKS_PER_TILE, CHUNK), jnp.int32),
        ],
    )(_deg_kernel)
    return k(dst2)


def _prop_kernel(y_hbm, src_hbm, dst_hbm, p0_hbm, p1_hbm,
                 acc, src_v, dst_v, rows0, rows1, gsem0, gsem1):
    c = lax.axis_index("c")
    s = lax.axis_index("s")
    wid = s * 2 + c
    rbase = s * ROWS_PER_TILE

    pltpu.sync_copy(y_hbm.at[pl.ds(rbase, ROWS_PER_TILE)],
                    acc.at[pl.ds(rbase, ROWS_PER_TILE)])
    plsc.subcore_barrier()

    def group(g, carry):
        base = wid * CHUNKS_PER_TILE + g * GK
        pltpu.sync_copy(src_hbm.at[pl.ds(base, GK)], src_v)
        pltpu.sync_copy(dst_hbm.at[pl.ds(base, GK)], dst_v)
        pltpu.async_copy(y_hbm.at[src_v.at[0]], rows0, gsem0)

        def body(i, carry2):
            j0 = 2 * i
            pltpu.make_async_copy(y_hbm.at[src_v.at[j0]], rows0, gsem0).wait()
            pltpu.async_copy(y_hbm.at[src_v.at[j0 + 1]], rows1, gsem1)
            pltpu.sync_copy(rows0, acc.at[dst_v.at[j0]], add=True)
            pltpu.make_async_copy(y_hbm.at[src_v.at[j0 + 1]], rows1,
                                  gsem1).wait()

            @pl.when(j0 + 2 < GK)
            def _():
                pltpu.async_copy(y_hbm.at[src_v.at[j0 + 2]], rows0, gsem0)
            pltpu.sync_copy(rows1, acc.at[dst_v.at[j0 + 1]], add=True)
            return carry2
        lax.fori_loop(0, GK // 2, body, 0)
        return carry
    lax.fori_loop(0, CHUNKS_PER_TILE // GK, group, 0)

    plsc.subcore_barrier()

    @pl.when(c == 0)
    def _():
        pltpu.sync_copy(acc.at[pl.ds(rbase, ROWS_PER_TILE)],
                        p0_hbm.at[pl.ds(rbase, ROWS_PER_TILE)])

    @pl.when(c == 1)
    def _():
        pltpu.sync_copy(acc.at[pl.ds(rbase, ROWS_PER_TILE)],
                        p1_hbm.at[pl.ds(rbase, ROWS_PER_TILE)])


def _prop2_kernel(ya_hbm, yb_hbm, src_hbm, dst_hbm, pa_hbm, pb_hbm,
                  acc, src_v, dst_v, rows0, rows1, gsem0, gsem1):
    c = lax.axis_index("c")
    s = lax.axis_index("s")
    rbase = s * ROWS_PER_TILE
    cpt2 = 2 * CHUNKS_PER_TILE

    def run(y_hbm, p_hbm):
        pltpu.sync_copy(y_hbm.at[pl.ds(rbase, ROWS_PER_TILE)],
                        acc.at[pl.ds(rbase, ROWS_PER_TILE)])
        plsc.subcore_barrier()

        def group(g, carry):
            base = s * cpt2 + g * GK
            pltpu.sync_copy(src_hbm.at[pl.ds(base, GK)], src_v)
            pltpu.sync_copy(dst_hbm.at[pl.ds(base, GK)], dst_v)
            pltpu.async_copy(y_hbm.at[src_v.at[0]], rows0, gsem0)

            def body(i, carry2):
                j0 = 2 * i
                pltpu.make_async_copy(y_hbm.at[src_v.at[j0]], rows0,
                                      gsem0).wait()
                pltpu.async_copy(y_hbm.at[src_v.at[j0 + 1]], rows1, gsem1)
                pltpu.sync_copy(rows0, acc.at[dst_v.at[j0]], add=True)
                pltpu.make_async_copy(y_hbm.at[src_v.at[j0 + 1]], rows1,
                                      gsem1).wait()

                @pl.when(j0 + 2 < GK)
                def _():
                    pltpu.async_copy(y_hbm.at[src_v.at[j0 + 2]], rows0, gsem0)
                pltpu.sync_copy(rows1, acc.at[dst_v.at[j0 + 1]], add=True)
                return carry2
            lax.fori_loop(0, GK // 2, body, 0)
            return carry
        lax.fori_loop(0, cpt2 // GK, group, 0)

        plsc.subcore_barrier()
        pltpu.sync_copy(acc.at[pl.ds(rbase, ROWS_PER_TILE)],
                        p_hbm.at[pl.ds(rbase, ROWS_PER_TILE)])

    @pl.when(c == 0)
    def _():
        run(ya_hbm, pa_hbm)

    @pl.when(c == 1)
    def _():
        run(yb_hbm, pb_hbm)


def _propagate2(ya, yb, src2, dst2):
    k = functools.partial(
        pl.kernel,
        mesh=_mesh,
        out_type=[jax.ShapeDtypeStruct((NPAD, 128), jnp.float32),
                  jax.ShapeDtypeStruct((NPAD, 128), jnp.float32)],
        scratch_types=[
            pltpu.VMEM_SHARED((NPAD, 128), jnp.float32),
            pltpu.VMEM((GK, CHUNK), jnp.int32),
            pltpu.VMEM((GK, CHUNK), jnp.int32),
            pltpu.VMEM((CHUNK, 128), jnp.float32),
            pltpu.VMEM((CHUNK, 128), jnp.float32),
            pltpu.SemaphoreType.DMA,
            pltpu.SemaphoreType.DMA,
        ],
    )(_prop2_kernel)
    return k(ya, yb, src2, dst2)


def _propagate(y, src2, dst2):
    k = functools.partial(
        pl.kernel,
        mesh=_mesh,
        out_type=[jax.ShapeDtypeStruct((NPAD, 128), jnp.float32),
                  jax.ShapeDtypeStruct((NPAD, 128), jnp.float32)],
        scratch_types=[
            pltpu.VMEM_SHARED((NPAD, 128), jnp.float32),
            pltpu.VMEM((GK, CHUNK), jnp.int32),
            pltpu.VMEM((GK, CHUNK), jnp.int32),
            pltpu.VMEM((CHUNK, 128), jnp.float32),
            pltpu.VMEM((CHUNK, 128), jnp.float32),
            pltpu.SemaphoreType.DMA,
            pltpu.SemaphoreType.DMA,
        ],
    )(_prop_kernel)
    return k(y, src2, dst2)


_ROWS = 1024
_GRID = NPAD // _ROWS


def _first_body(x_ref, w_ref, d0_ref, d1_ref, y_ref, dinv_ref):
    dinv = lax.rsqrt(d0_ref[...] + d1_ref[...] + 1.0)
    y_ref[...] = dinv * jnp.dot(x_ref[...], w_ref[...],
                                preferred_element_type=jnp.float32)
    dinv_ref[...] = dinv


def _first(x_pad, W1, d0, d1):
    rb = lambda i: (i, 0)
    return pl.pallas_call(
        _first_body,
        grid=(_GRID,),
        in_specs=[
            pl.BlockSpec((_ROWS, 128), rb),
            pl.BlockSpec((128, 128), lambda i: (0, 0)),
            pl.BlockSpec((_ROWS, 1), rb),
            pl.BlockSpec((_ROWS, 1), rb),
        ],
        out_specs=[pl.BlockSpec((_ROWS, 128), rb),
                   pl.BlockSpec((_ROWS, 1), rb)],
        out_shape=[jax.ShapeDtypeStruct((NPAD, 128), jnp.float32),
                   jax.ShapeDtypeStruct((NPAD, 1), jnp.float32)],
    )(x_pad, W1, d0.reshape(NPAD, 1), d1.reshape(NPAD, 1))


def _comb1_body(p0, p1, y, dinv, w_ref, b_ref, out_ref):
    s = p0[...] + p1[...] - y[...]
    h = jnp.maximum(dinv[...] * s + b_ref[...], 0.0)
    out_ref[...] = dinv[...] * jnp.dot(h, w_ref[...],
                                       preferred_element_type=jnp.float32)


def _comb1(p0, p1, y1, dinv, W2, b1):
    rb = lambda i: (i, 0)
    return pl.pallas_call(
        _comb1_body,
        grid=(_GRID,),
        in_specs=[
            pl.BlockSpec((_ROWS, 128), rb),
            pl.BlockSpec((_ROWS, 128), rb),
            pl.BlockSpec((_ROWS, 128), rb),
            pl.BlockSpec((_ROWS, 1), rb),
            pl.BlockSpec((128, 256), lambda i: (0, 0)),
            pl.BlockSpec((1, 128), lambda i: (0, 0)),
        ],
        out_specs=pl.BlockSpec((_ROWS, 256), rb),
        out_shape=jax.ShapeDtypeStruct((NPAD, 256), jnp.float32),
    )(p0, p1, y1, dinv, W2, b1.reshape(1, 128))


def _comb2_body(qa, qb, dinv, w_ref, b_ref, out_ref):
    s = jnp.concatenate([qa[...], qb[...]], axis=1)
    h = jnp.maximum(dinv[...] * s + b_ref[...], 0.0)
    out_ref[...] = dinv[...] * jnp.dot(h, w_ref[...],
                                       preferred_element_type=jnp.float32)


def _comb2(qa, qb, dinv, W3, b2):
    rb = lambda i: (i, 0)
    return pl.pallas_call(
        _comb2_body,
        grid=(_GRID,),
        in_specs=[
            pl.BlockSpec((_ROWS, 128), rb),
            pl.BlockSpec((_ROWS, 128), rb),
            pl.BlockSpec((_ROWS, 1), rb),
            pl.BlockSpec((256, 128), lambda i: (0, 0)),
            pl.BlockSpec((1, 256), lambda i: (0, 0)),
        ],
        out_specs=pl.BlockSpec((_ROWS, 128), rb),
        out_shape=jax.ShapeDtypeStruct((NPAD, 128), jnp.float32),
    )(qa, qb, dinv, W3, b2.reshape(1, 256))


def _final_body(r0, r1, y3, dinv, b_ref, out_ref):
    out_ref[...] = dinv[...] * (r0[...] + r1[...] - y3[...]) + b_ref[...]


def _final(r0, r1, y3, dinv, b3):
    rb = lambda i: (i, 0)
    rows = 1000
    return pl.pallas_call(
        _final_body,
        grid=(N // rows,),
        in_specs=[
            pl.BlockSpec((rows, 128), rb),
            pl.BlockSpec((rows, 128), rb),
            pl.BlockSpec((rows, 128), rb),
            pl.BlockSpec((rows, 1), rb),
            pl.BlockSpec((1, 128), lambda i: (0, 0)),
        ],
        out_specs=pl.BlockSpec((rows, 128), rb),
        out_shape=jax.ShapeDtypeStruct((N, 128), jnp.float32),
    )(r0, r1, y3, dinv, b3.reshape(1, 128))


def kernel(x, edge_index, W1, b1, W2, b2, W3, b3):
    src = edge_index[0].astype(jnp.int32)
    dst = edge_index[1].astype(jnp.int32)
    pad = EPAD - E
    padi = jnp.arange(pad, dtype=jnp.int32)
    srcp = jnp.concatenate([src, padi % N])
    dstp = jnp.concatenate([dst, N + padi % (NPAD - N)])
    src2 = srcp.reshape(EPAD // CHUNK, CHUNK)
    dst2 = dstp.reshape(EPAD // CHUNK, CHUNK)
    x_pad = jnp.pad(x, ((0, NPAD - N), (0, 0)))

    d0, d1 = _degrees(dst2)
    y1, dinv = _first(x_pad, W1, d0, d1)

    p0, p1 = _propagate(y1, src2, dst2)
    y2 = _comb1(p0, p1, y1, dinv, W2, b1)

    qa, qb = _propagate2(y2[:, :128], y2[:, 128:], src2, dst2)
    y3 = _comb2(qa, qb, dinv, W3, b2)

    r0, r1 = _propagate(y3, src2, dst2)
    return _final(r0, r1, y3, dinv, b3)
